# baseline (device time: 698624 ns/iter reference)
import jax
import jax.numpy as jnp
from jax import lax
from jax.experimental import pallas as pl
from jax.experimental.pallas import tpu as pltpu

N_DEV = 4
S_SHARD = 1024
S_FULL = 4096
D = 1024
HQ = 8
DH = 128
QBLK = 128
KBLK = 512
SCALE = 0.08838834764831843
LOG2E = 1.4426950408889634
F32 = jnp.float32


def _neighbor_barrier(left, right):
    barrier = pltpu.get_barrier_semaphore()
    for nbr in (left, right):
        pl.semaphore_signal(
            barrier, inc=1,
            device_id=(nbr,), device_id_type=pl.DeviceIdType.MESH,
        )
    pl.semaphore_wait(barrier, 2)


def _ag_body(x_ref, out_ref, send_sems, recv_sems):
    my = lax.axis_index("i")
    left = (my + N_DEV - 1) % N_DEV
    right = (my + 1) % N_DEV
    _neighbor_barrier(left, right)

    out_ref[pl.ds(my * S_SHARD, S_SHARD), :] = x_ref[0]

    for h in range(N_DEV - 1):
        origin = (my + N_DEV - h) % N_DEV
        sl = pl.ds(origin * S_SHARD, S_SHARD)
        rdma = pltpu.make_async_remote_copy(
            src_ref=out_ref.at[sl, :],
            dst_ref=out_ref.at[sl, :],
            send_sem=send_sems.at[h],
            recv_sem=recv_sems.at[h],
            device_id=(right,),
            device_id_type=pl.DeviceIdType.MESH,
        )
        rdma.start()
        rdma.wait()


def _ring_all_gather(x):
    return pl.pallas_call(
        _ag_body,
        out_shape=jax.ShapeDtypeStruct((S_FULL, D), x.dtype),
        in_specs=[pl.BlockSpec(memory_space=pltpu.VMEM)],
        out_specs=pl.BlockSpec(memory_space=pltpu.VMEM),
        scratch_shapes=[
            pltpu.SemaphoreType.DMA((N_DEV - 1,)),
            pltpu.SemaphoreType.DMA((N_DEV - 1,)),
        ],
        compiler_params=pltpu.CompilerParams(
            collective_id=0, vmem_limit_bytes=64 * 1024 * 1024,
        ),
    )(x)


def _attn_body(x_ref, wq_ref, wk_ref, wv_ref, wo_ref, cos_ref, sin_ref,
               out_ref, krot_ref, vh_ref):
    h = pl.program_id(0)

    ii = lax.broadcasted_iota(jnp.int32, (DH, DH), 0)
    jj = lax.broadcasted_iota(jnp.int32, (DH, DH), 1)
    rmat = jnp.where(
        (ii % 2 == 0) & (jj == ii + 1), 1.0,
        jnp.where((ii % 2 == 1) & (jj == ii - 1), -1.0, 0.0),
    ).astype(F32)

    BF16 = jnp.bfloat16
    wk16 = wk_ref[...].astype(BF16)
    wv16 = wv_ref[...].astype(BF16)
    wq16 = wq_ref[...].astype(BF16)
    wo16 = wo_ref[...].astype(BF16)
    rmat16 = rmat.astype(BF16)

    def kstep(c, carry):
        rows = pl.ds(c * KBLK, KBLK)
        xb = x_ref[rows, :]
        kh = jnp.dot(xb, wk16, preferred_element_type=F32)
        krot_ref[rows, :] = (
            kh * cos_ref[rows, :]
            + jnp.dot(kh.astype(BF16), rmat16, preferred_element_type=F32)
            * sin_ref[rows, :]
        ).astype(BF16)
        vh_ref[rows, :DH] = jnp.dot(
            xb, wv16, preferred_element_type=F32
        ).astype(BF16)
        lane = lax.broadcasted_iota(jnp.int32, (KBLK, DH), 1)
        vh_ref[rows, DH:] = jnp.where(lane == 0, 1.0, 0.0).astype(BF16)
        return carry

    lax.fori_loop(0, S_FULL // KBLK, kstep, 0)

    def qstep(qb, carry):
        rows = pl.ds(qb * QBLK, QBLK)
        xb = x_ref[rows, :]
        qh = jnp.dot(xb, wq16, preferred_element_type=F32)
        qrot = (qh * cos_ref[rows, :]
                + jnp.dot(qh.astype(BF16), rmat16, preferred_element_type=F32)
                * sin_ref[rows, :])
        q16 = (qrot * (SCALE * LOG2E)).astype(BF16)
        s = lax.dot_general(
            q16, krot_ref[...], (((1,), (1,)), ((), ())),
            preferred_element_type=F32,
        )
        e = jnp.exp2(s).astype(BF16)
        ctxsum = jnp.dot(e, vh_ref[...], preferred_element_type=F32)
        ctx = ctxsum[:, :DH] * (1.0 / ctxsum[:, DH:DH + 1])
        contrib = jnp.dot(ctx.astype(BF16), wo16, preferred_element_type=F32)

        @pl.when(h == 0)
        def _():
            out_ref[rows, :] = contrib

        @pl.when(h != 0)
        def _():
            out_ref[rows, :] = out_ref[rows, :] + contrib

        return carry

    lax.fori_loop(0, S_FULL // QBLK, qstep, 0)


def _attention(x_full, Wq, Wk, Wv, Wo, cos, sin):
    return pl.pallas_call(
        _attn_body,
        grid=(HQ,),
        out_shape=jax.ShapeDtypeStruct((S_FULL, D), F32),
        in_specs=[
            pl.BlockSpec(memory_space=pltpu.VMEM),
            pl.BlockSpec((D, DH), lambda h: (0, h)),
            pl.BlockSpec((D, DH), lambda h: (0, h)),
            pl.BlockSpec((D, DH), lambda h: (0, h)),
            pl.BlockSpec((DH, D), lambda h: (h, 0)),
            pl.BlockSpec(memory_space=pltpu.VMEM),
            pl.BlockSpec(memory_space=pltpu.VMEM),
        ],
        out_specs=pl.BlockSpec(memory_space=pltpu.VMEM),
        scratch_shapes=[
            pltpu.VMEM((S_FULL, DH), jnp.bfloat16),
            pltpu.VMEM((S_FULL, 2 * DH), jnp.bfloat16),
        ],
        compiler_params=pltpu.CompilerParams(
            vmem_limit_bytes=64 * 1024 * 1024,
        ),
    )(x_full, Wq, Wk, Wv, Wo, cos, sin)


def _rs_body(p_ref, out_ref, recv_ref, send_sems, recv_sems):
    my = lax.axis_index("i")
    left = (my + N_DEV - 1) % N_DEV
    right = (my + 1) % N_DEV
    _neighbor_barrier(left, right)

    c0 = (my + N_DEV - 1) % N_DEV
    rdma = pltpu.make_async_remote_copy(
        src_ref=p_ref.at[pl.ds(c0 * S_SHARD, S_SHARD), :],
        dst_ref=recv_ref.at[0],
        send_sem=send_sems.at[0],
        recv_sem=recv_sems.at[0],
        device_id=(right,),
        device_id_type=pl.DeviceIdType.MESH,
    )
    rdma.start()
    rdma.wait()

    for s in range(1, N_DEV - 1):
        c = (my + N_DEV - 1 - s) % N_DEV
        recv_ref[s - 1, :, :] = (
            recv_ref[s - 1, :, :].astype(F32)
            + p_ref[pl.ds(c * S_SHARD, S_SHARD), :].astype(F32)
        ).astype(recv_ref.dtype)
        rdma = pltpu.make_async_remote_copy(
            src_ref=recv_ref.at[s - 1],
            dst_ref=recv_ref.at[s],
            send_sem=send_sems.at[s],
            recv_sem=recv_sems.at[s],
            device_id=(right,),
            device_id_type=pl.DeviceIdType.MESH,
        )
        rdma.start()
        rdma.wait()

    out_ref[0] = (
        recv_ref[N_DEV - 2, :, :].astype(F32)
        + p_ref[pl.ds(my * S_SHARD, S_SHARD), :].astype(F32)
    )


def _reduce_scatter(partial):
    return pl.pallas_call(
        _rs_body,
        out_shape=jax.ShapeDtypeStruct((1, S_SHARD, D), F32),
        in_specs=[pl.BlockSpec(memory_space=pltpu.VMEM)],
        out_specs=pl.BlockSpec(memory_space=pltpu.VMEM),
        scratch_shapes=[
            pltpu.VMEM((N_DEV - 1, S_SHARD, D), partial.dtype),
            pltpu.SemaphoreType.DMA((N_DEV - 1,)),
            pltpu.SemaphoreType.DMA((N_DEV - 1,)),
        ],
        compiler_params=pltpu.CompilerParams(
            collective_id=1, vmem_limit_bytes=64 * 1024 * 1024,
        ),
    )(partial)


def kernel(x, Wq, Wk, Wv, Wo):
    inv = 1.0 / (10000.0 ** (jnp.arange(0, DH, 2, dtype=F32) / DH))
    pos = jnp.arange(S_FULL, dtype=F32)[:, None] * inv[None, :]
    cos = jnp.repeat(jnp.cos(pos), 2, axis=-1)
    sin = jnp.repeat(jnp.sin(pos), 2, axis=-1)

    x_full = _ring_all_gather(x.astype(jnp.bfloat16))
    partial = _attention(x_full, Wq, Wk, Wv, Wo, cos, sin)
    return _reduce_scatter(partial.astype(jnp.bfloat16))


# device time: 576848 ns/iter; 1.2111x vs baseline; 1.2111x over previous
import jax
import jax.numpy as jnp
from jax import lax
from jax.experimental import pallas as pl
from jax.experimental.pallas import tpu as pltpu

N_DEV = 4
S_SHARD = 1024
S_FULL = 4096
D = 1024
HQ = 8
DH = 128
QBLK = 512
KBLK = 512
SCALE = 0.08838834764831843
LOG2E = 1.4426950408889634
F32 = jnp.float32


def _neighbor_barrier(left, right):
    barrier = pltpu.get_barrier_semaphore()
    for nbr in (left, right):
        pl.semaphore_signal(
            barrier, inc=1,
            device_id=(nbr,), device_id_type=pl.DeviceIdType.MESH,
        )
    pl.semaphore_wait(barrier, 2)


def _ag_body(x_ref, out_ref, send_sems, recv_sems):
    my = lax.axis_index("i")
    left = (my + N_DEV - 1) % N_DEV
    right = (my + 1) % N_DEV
    _neighbor_barrier(left, right)

    out_ref[pl.ds(my * S_SHARD, S_SHARD), :] = x_ref[0]

    for h in range(N_DEV - 1):
        origin = (my + N_DEV - h) % N_DEV
        sl = pl.ds(origin * S_SHARD, S_SHARD)
        rdma = pltpu.make_async_remote_copy(
            src_ref=out_ref.at[sl, :],
            dst_ref=out_ref.at[sl, :],
            send_sem=send_sems.at[h],
            recv_sem=recv_sems.at[h],
            device_id=(right,),
            device_id_type=pl.DeviceIdType.MESH,
        )
        rdma.start()
        rdma.wait()


def _ring_all_gather(x):
    return pl.pallas_call(
        _ag_body,
        out_shape=jax.ShapeDtypeStruct((S_FULL, D), x.dtype),
        in_specs=[pl.BlockSpec(memory_space=pltpu.VMEM)],
        out_specs=pl.BlockSpec(memory_space=pltpu.VMEM),
        scratch_shapes=[
            pltpu.SemaphoreType.DMA((N_DEV - 1,)),
            pltpu.SemaphoreType.DMA((N_DEV - 1,)),
        ],
        compiler_params=pltpu.CompilerParams(
            collective_id=0, vmem_limit_bytes=64 * 1024 * 1024,
        ),
    )(x)


def _attn_body(x_ref, wq_ref, wk_ref, wv_ref, wo_ref, cos_ref, sin_ref,
               out_ref, krot_ref, vh_ref):
    h = pl.program_id(0)

    ii = lax.broadcasted_iota(jnp.int32, (DH, DH), 0)
    jj = lax.broadcasted_iota(jnp.int32, (DH, DH), 1)
    rmat = jnp.where(
        (ii % 2 == 0) & (jj == ii + 1), 1.0,
        jnp.where((ii % 2 == 1) & (jj == ii - 1), -1.0, 0.0),
    ).astype(F32)

    BF16 = jnp.bfloat16
    wk16 = wk_ref[...].astype(BF16)
    wv16 = wv_ref[...].astype(BF16)
    wq16 = wq_ref[...].astype(BF16)
    wo16 = wo_ref[...].astype(BF16)
    rmat16 = rmat.astype(BF16)

    def kstep(c, carry):
        rows = pl.ds(c * KBLK, KBLK)
        xb = x_ref[rows, :]
        kh = jnp.dot(xb, wk16, preferred_element_type=F32)
        krot_ref[rows, :] = (
            kh * cos_ref[rows, :]
            + jnp.dot(kh.astype(BF16), rmat16, preferred_element_type=F32)
            * sin_ref[rows, :]
        ).astype(BF16)
        vh_ref[rows, :DH] = jnp.dot(
            xb, wv16, preferred_element_type=F32
        ).astype(BF16)
        lane = lax.broadcasted_iota(jnp.int32, (KBLK, DH), 1)
        vh_ref[rows, DH:] = jnp.where(lane == 0, 1.0, 0.0).astype(BF16)
        return carry

    lax.fori_loop(0, S_FULL // KBLK, kstep, 0)

    def qstep(qb, carry):
        rows = pl.ds(qb * QBLK, QBLK)
        xb = x_ref[rows, :]
        qh = jnp.dot(xb, wq16, preferred_element_type=F32)
        qrot = (qh * cos_ref[rows, :]
                + jnp.dot(qh.astype(BF16), rmat16, preferred_element_type=F32)
                * sin_ref[rows, :])
        q16 = (qrot * (SCALE * LOG2E)).astype(BF16)
        s = lax.dot_general(
            q16, krot_ref[...], (((1,), (1,)), ((), ())),
            preferred_element_type=F32,
        )
        e = jnp.exp2(s).astype(BF16)
        ctxsum = jnp.dot(e, vh_ref[...], preferred_element_type=F32)
        ctx = ctxsum[:, :DH] * (1.0 / ctxsum[:, DH:DH + 1])
        contrib = jnp.dot(ctx.astype(BF16), wo16, preferred_element_type=F32)

        @pl.when(h == 0)
        def _():
            out_ref[rows, :] = contrib

        @pl.when(h != 0)
        def _():
            out_ref[rows, :] = out_ref[rows, :] + contrib

        return carry

    lax.fori_loop(0, S_FULL // QBLK, qstep, 0)


def _attention(x_full, Wq, Wk, Wv, Wo, cos, sin):
    return pl.pallas_call(
        _attn_body,
        grid=(HQ,),
        out_shape=jax.ShapeDtypeStruct((S_FULL, D), F32),
        in_specs=[
            pl.BlockSpec(memory_space=pltpu.VMEM),
            pl.BlockSpec((D, DH), lambda h: (0, h)),
            pl.BlockSpec((D, DH), lambda h: (0, h)),
            pl.BlockSpec((D, DH), lambda h: (0, h)),
            pl.BlockSpec((DH, D), lambda h: (h, 0)),
            pl.BlockSpec(memory_space=pltpu.VMEM),
            pl.BlockSpec(memory_space=pltpu.VMEM),
        ],
        out_specs=pl.BlockSpec(memory_space=pltpu.VMEM),
        scratch_shapes=[
            pltpu.VMEM((S_FULL, DH), jnp.bfloat16),
            pltpu.VMEM((S_FULL, 2 * DH), jnp.bfloat16),
        ],
        compiler_params=pltpu.CompilerParams(
            vmem_limit_bytes=64 * 1024 * 1024,
        ),
    )(x_full, Wq, Wk, Wv, Wo, cos, sin)


def _rs_body(p_ref, out_ref, recv_ref, send_sems, recv_sems):
    my = lax.axis_index("i")
    left = (my + N_DEV - 1) % N_DEV
    right = (my + 1) % N_DEV
    _neighbor_barrier(left, right)

    c0 = (my + N_DEV - 1) % N_DEV
    rdma = pltpu.make_async_remote_copy(
        src_ref=p_ref.at[pl.ds(c0 * S_SHARD, S_SHARD), :],
        dst_ref=recv_ref.at[0],
        send_sem=send_sems.at[0],
        recv_sem=recv_sems.at[0],
        device_id=(right,),
        device_id_type=pl.DeviceIdType.MESH,
    )
    rdma.start()
    rdma.wait()

    for s in range(1, N_DEV - 1):
        c = (my + N_DEV - 1 - s) % N_DEV
        recv_ref[s - 1, :, :] = (
            recv_ref[s - 1, :, :].astype(F32)
            + p_ref[pl.ds(c * S_SHARD, S_SHARD), :].astype(F32)
        ).astype(recv_ref.dtype)
        rdma = pltpu.make_async_remote_copy(
            src_ref=recv_ref.at[s - 1],
            dst_ref=recv_ref.at[s],
            send_sem=send_sems.at[s],
            recv_sem=recv_sems.at[s],
            device_id=(right,),
            device_id_type=pl.DeviceIdType.MESH,
        )
        rdma.start()
        rdma.wait()

    out_ref[0] = (
        recv_ref[N_DEV - 2, :, :].astype(F32)
        + p_ref[pl.ds(my * S_SHARD, S_SHARD), :].astype(F32)
    )


def _reduce_scatter(partial):
    return pl.pallas_call(
        _rs_body,
        out_shape=jax.ShapeDtypeStruct((1, S_SHARD, D), F32),
        in_specs=[pl.BlockSpec(memory_space=pltpu.VMEM)],
        out_specs=pl.BlockSpec(memory_space=pltpu.VMEM),
        scratch_shapes=[
            pltpu.VMEM((N_DEV - 1, S_SHARD, D), partial.dtype),
            pltpu.SemaphoreType.DMA((N_DEV - 1,)),
            pltpu.SemaphoreType.DMA((N_DEV - 1,)),
        ],
        compiler_params=pltpu.CompilerParams(
            collective_id=1, vmem_limit_bytes=64 * 1024 * 1024,
        ),
    )(partial)


def kernel(x, Wq, Wk, Wv, Wo):
    inv = 1.0 / (10000.0 ** (jnp.arange(0, DH, 2, dtype=F32) / DH))
    pos = jnp.arange(S_FULL, dtype=F32)[:, None] * inv[None, :]
    cos = jnp.repeat(jnp.cos(pos), 2, axis=-1)
    sin = jnp.repeat(jnp.sin(pos), 2, axis=-1)

    x_full = _ring_all_gather(x.astype(jnp.bfloat16))
    partial = _attention(x_full, Wq, Wk, Wv, Wo, cos, sin)
    return _reduce_scatter(partial.astype(jnp.bfloat16))


# device time: 559260 ns/iter; 1.2492x vs baseline; 1.0314x over previous
import jax
import jax.numpy as jnp
from jax import lax
from jax.experimental import pallas as pl
from jax.experimental.pallas import tpu as pltpu

N_DEV = 4
S_SHARD = 1024
S_FULL = 4096
D = 1024
HQ = 8
DH = 128
QBLK = 1024
KBLK = 512
SCALE = 0.08838834764831843
LOG2E = 1.4426950408889634
F32 = jnp.float32


def _neighbor_barrier(left, right):
    barrier = pltpu.get_barrier_semaphore()
    for nbr in (left, right):
        pl.semaphore_signal(
            barrier, inc=1,
            device_id=(nbr,), device_id_type=pl.DeviceIdType.MESH,
        )
    pl.semaphore_wait(barrier, 2)


def _ag_body(x_ref, out_ref, send_sems, recv_sems):
    my = lax.axis_index("i")
    left = (my + N_DEV - 1) % N_DEV
    right = (my + 1) % N_DEV
    _neighbor_barrier(left, right)

    out_ref[pl.ds(my * S_SHARD, S_SHARD), :] = x_ref[0]

    for h in range(N_DEV - 1):
        origin = (my + N_DEV - h) % N_DEV
        sl = pl.ds(origin * S_SHARD, S_SHARD)
        rdma = pltpu.make_async_remote_copy(
            src_ref=out_ref.at[sl, :],
            dst_ref=out_ref.at[sl, :],
            send_sem=send_sems.at[h],
            recv_sem=recv_sems.at[h],
            device_id=(right,),
            device_id_type=pl.DeviceIdType.MESH,
        )
        rdma.start()
        rdma.wait()


def _ring_all_gather(x):
    return pl.pallas_call(
        _ag_body,
        out_shape=jax.ShapeDtypeStruct((S_FULL, D), x.dtype),
        in_specs=[pl.BlockSpec(memory_space=pltpu.VMEM)],
        out_specs=pl.BlockSpec(memory_space=pltpu.VMEM),
        scratch_shapes=[
            pltpu.SemaphoreType.DMA((N_DEV - 1,)),
            pltpu.SemaphoreType.DMA((N_DEV - 1,)),
        ],
        compiler_params=pltpu.CompilerParams(
            collective_id=0, vmem_limit_bytes=64 * 1024 * 1024,
        ),
    )(x)


def _attn_body(x_ref, wq_ref, wk_ref, wv_ref, wo_ref, cos_ref, sin_ref,
               out_ref, krot_ref, vh_ref):
    h = pl.program_id(0)

    ii = lax.broadcasted_iota(jnp.int32, (DH, DH), 0)
    jj = lax.broadcasted_iota(jnp.int32, (DH, DH), 1)
    rmat = jnp.where(
        (ii % 2 == 0) & (jj == ii + 1), 1.0,
        jnp.where((ii % 2 == 1) & (jj == ii - 1), -1.0, 0.0),
    ).astype(F32)

    BF16 = jnp.bfloat16
    wk16 = wk_ref[...].astype(BF16)
    wv16 = wv_ref[...].astype(BF16)
    wq16 = wq_ref[...].astype(BF16)
    wo16 = wo_ref[...].astype(BF16)
    rmat16 = rmat.astype(BF16)

    def kstep(c, carry):
        rows = pl.ds(c * KBLK, KBLK)
        xb = x_ref[rows, :]
        kh = jnp.dot(xb, wk16, preferred_element_type=F32)
        krot_ref[rows, :] = (
            kh * cos_ref[rows, :]
            + jnp.dot(kh.astype(BF16), rmat16, preferred_element_type=F32)
            * sin_ref[rows, :]
        ).astype(BF16)
        vh_ref[rows, :DH] = jnp.dot(
            xb, wv16, preferred_element_type=F32
        ).astype(BF16)
        lane = lax.broadcasted_iota(jnp.int32, (KBLK, DH), 1)
        vh_ref[rows, DH:] = jnp.where(lane == 0, 1.0, 0.0).astype(BF16)
        return carry

    lax.fori_loop(0, S_FULL // KBLK, kstep, 0)

    def qstep(qb, carry):
        rows = pl.ds(qb * QBLK, QBLK)
        xb = x_ref[rows, :]
        qh = jnp.dot(xb, wq16, preferred_element_type=F32)
        qrot = (qh * cos_ref[rows, :]
                + jnp.dot(qh.astype(BF16), rmat16, preferred_element_type=F32)
                * sin_ref[rows, :])
        q16 = (qrot * (SCALE * LOG2E)).astype(BF16)
        s = lax.dot_general(
            q16, krot_ref[...], (((1,), (1,)), ((), ())),
            preferred_element_type=F32,
        )
        e = jnp.exp2(s).astype(BF16)
        ctxsum = jnp.dot(e, vh_ref[...], preferred_element_type=F32)
        ctx = ctxsum[:, :DH] * (1.0 / ctxsum[:, DH:DH + 1])
        contrib = jnp.dot(ctx.astype(BF16), wo16, preferred_element_type=F32)

        @pl.when(h == 0)
        def _():
            out_ref[rows, :] = contrib

        @pl.when(h != 0)
        def _():
            out_ref[rows, :] = out_ref[rows, :] + contrib

        return carry

    lax.fori_loop(0, S_FULL // QBLK, qstep, 0)


def _attention(x_full, Wq, Wk, Wv, Wo, cos, sin):
    return pl.pallas_call(
        _attn_body,
        grid=(HQ,),
        out_shape=jax.ShapeDtypeStruct((S_FULL, D), F32),
        in_specs=[
            pl.BlockSpec(memory_space=pltpu.VMEM),
            pl.BlockSpec((D, DH), lambda h: (0, h)),
            pl.BlockSpec((D, DH), lambda h: (0, h)),
            pl.BlockSpec((D, DH), lambda h: (0, h)),
            pl.BlockSpec((DH, D), lambda h: (h, 0)),
            pl.BlockSpec(memory_space=pltpu.VMEM),
            pl.BlockSpec(memory_space=pltpu.VMEM),
        ],
        out_specs=pl.BlockSpec(memory_space=pltpu.VMEM),
        scratch_shapes=[
            pltpu.VMEM((S_FULL, DH), jnp.bfloat16),
            pltpu.VMEM((S_FULL, 2 * DH), jnp.bfloat16),
        ],
        compiler_params=pltpu.CompilerParams(
            vmem_limit_bytes=64 * 1024 * 1024,
        ),
    )(x_full, Wq, Wk, Wv, Wo, cos, sin)


def _rs_body(p_ref, out_ref, recv_ref, send_sems, recv_sems):
    my = lax.axis_index("i")
    left = (my + N_DEV - 1) % N_DEV
    right = (my + 1) % N_DEV
    _neighbor_barrier(left, right)

    c0 = (my + N_DEV - 1) % N_DEV
    rdma = pltpu.make_async_remote_copy(
        src_ref=p_ref.at[pl.ds(c0 * S_SHARD, S_SHARD), :],
        dst_ref=recv_ref.at[0],
        send_sem=send_sems.at[0],
        recv_sem=recv_sems.at[0],
        device_id=(right,),
        device_id_type=pl.DeviceIdType.MESH,
    )
    rdma.start()
    rdma.wait()

    for s in range(1, N_DEV - 1):
        c = (my + N_DEV - 1 - s) % N_DEV
        recv_ref[s - 1, :, :] = (
            recv_ref[s - 1, :, :].astype(F32)
            + p_ref[pl.ds(c * S_SHARD, S_SHARD), :].astype(F32)
        ).astype(recv_ref.dtype)
        rdma = pltpu.make_async_remote_copy(
            src_ref=recv_ref.at[s - 1],
            dst_ref=recv_ref.at[s],
            send_sem=send_sems.at[s],
            recv_sem=recv_sems.at[s],
            device_id=(right,),
            device_id_type=pl.DeviceIdType.MESH,
        )
        rdma.start()
        rdma.wait()

    out_ref[0] = (
        recv_ref[N_DEV - 2, :, :].astype(F32)
        + p_ref[pl.ds(my * S_SHARD, S_SHARD), :].astype(F32)
    )


def _reduce_scatter(partial):
    return pl.pallas_call(
        _rs_body,
        out_shape=jax.ShapeDtypeStruct((1, S_SHARD, D), F32),
        in_specs=[pl.BlockSpec(memory_space=pltpu.VMEM)],
        out_specs=pl.BlockSpec(memory_space=pltpu.VMEM),
        scratch_shapes=[
            pltpu.VMEM((N_DEV - 1, S_SHARD, D), partial.dtype),
            pltpu.SemaphoreType.DMA((N_DEV - 1,)),
            pltpu.SemaphoreType.DMA((N_DEV - 1,)),
        ],
        compiler_params=pltpu.CompilerParams(
            collective_id=1, vmem_limit_bytes=64 * 1024 * 1024,
        ),
    )(partial)


def kernel(x, Wq, Wk, Wv, Wo):
    inv = 1.0 / (10000.0 ** (jnp.arange(0, DH, 2, dtype=F32) / DH))
    pos = jnp.arange(S_FULL, dtype=F32)[:, None] * inv[None, :]
    cos = jnp.repeat(jnp.cos(pos), 2, axis=-1)
    sin = jnp.repeat(jnp.sin(pos), 2, axis=-1)

    x_full = _ring_all_gather(x.astype(jnp.bfloat16))
    partial = _attention(x_full, Wq, Wk, Wv, Wo, cos, sin)
    return _reduce_scatter(partial.astype(jnp.bfloat16))


# device time: 492412 ns/iter; 1.4188x vs baseline; 1.1358x over previous
import jax
import jax.numpy as jnp
from jax import lax
from jax.experimental import pallas as pl
from jax.experimental.pallas import tpu as pltpu

N_DEV = 4
S_SHARD = 1024
S_FULL = 4096
D = 1024
HQ = 8
DH = 128
QBLK = 1024
KBLK = 512
SCALE = 0.08838834764831843
LOG2E = 1.4426950408889634
F32 = jnp.float32


def _neighbor_barrier(left, right):
    barrier = pltpu.get_barrier_semaphore()
    for nbr in (left, right):
        pl.semaphore_signal(
            barrier, inc=1,
            device_id=(nbr,), device_id_type=pl.DeviceIdType.MESH,
        )
    pl.semaphore_wait(barrier, 2)


HALF = S_SHARD // 2


def _ag_body(x_ref, out_ref, sa_sems, ra_sems, sb_sems, rb_sems):
    my = lax.axis_index("i")
    left = (my + N_DEV - 1) % N_DEV
    right = (my + 1) % N_DEV
    _neighbor_barrier(left, right)

    out_ref[pl.ds(my * S_SHARD, S_SHARD), :] = x_ref[0]

    for h in range(N_DEV - 1):
        oa = (my + N_DEV - h) % N_DEV
        ob = (my + h) % N_DEV
        sla = pl.ds(oa * S_SHARD, HALF)
        slb = pl.ds(ob * S_SHARD + HALF, HALF)
        ra = pltpu.make_async_remote_copy(
            src_ref=out_ref.at[sla, :], dst_ref=out_ref.at[sla, :],
            send_sem=sa_sems.at[h], recv_sem=ra_sems.at[h],
            device_id=(right,), device_id_type=pl.DeviceIdType.MESH,
        )
        rb = pltpu.make_async_remote_copy(
            src_ref=out_ref.at[slb, :], dst_ref=out_ref.at[slb, :],
            send_sem=sb_sems.at[h], recv_sem=rb_sems.at[h],
            device_id=(left,), device_id_type=pl.DeviceIdType.MESH,
        )
        ra.start()
        rb.start()
        ra.wait()
        rb.wait()


def _ring_all_gather(x):
    return pl.pallas_call(
        _ag_body,
        out_shape=jax.ShapeDtypeStruct((S_FULL, D), x.dtype),
        in_specs=[pl.BlockSpec(memory_space=pltpu.VMEM)],
        out_specs=pl.BlockSpec(memory_space=pltpu.VMEM),
        scratch_shapes=[
            pltpu.SemaphoreType.DMA((N_DEV - 1,)),
            pltpu.SemaphoreType.DMA((N_DEV - 1,)),
            pltpu.SemaphoreType.DMA((N_DEV - 1,)),
            pltpu.SemaphoreType.DMA((N_DEV - 1,)),
        ],
        compiler_params=pltpu.CompilerParams(
            collective_id=0, vmem_limit_bytes=64 * 1024 * 1024,
        ),
    )(x)


def _attn_body(x_ref, wq_ref, wk_ref, wv_ref, wo_ref, cos_ref, sin_ref,
               out_ref, krot_ref, vh_ref):
    h = pl.program_id(0)

    ii = lax.broadcasted_iota(jnp.int32, (DH, DH), 0)
    jj = lax.broadcasted_iota(jnp.int32, (DH, DH), 1)
    rmat = jnp.where(
        (ii % 2 == 0) & (jj == ii + 1), 1.0,
        jnp.where((ii % 2 == 1) & (jj == ii - 1), -1.0, 0.0),
    ).astype(F32)

    BF16 = jnp.bfloat16
    wk16 = wk_ref[...].astype(BF16)
    wv16 = wv_ref[...].astype(BF16)
    wq16 = wq_ref[...].astype(BF16)
    wo16 = wo_ref[...].astype(BF16)
    rmat16 = rmat.astype(BF16)

    def kstep(c, carry):
        rows = pl.ds(c * KBLK, KBLK)
        xb = x_ref[rows, :]
        kh = jnp.dot(xb, wk16, preferred_element_type=F32)
        krot_ref[rows, :] = (
            kh * cos_ref[rows, :]
            + jnp.dot(kh.astype(BF16), rmat16, preferred_element_type=F32)
            * sin_ref[rows, :]
        ).astype(BF16)
        vh_ref[rows, :DH] = jnp.dot(
            xb, wv16, preferred_element_type=F32
        ).astype(BF16)
        lane = lax.broadcasted_iota(jnp.int32, (KBLK, DH), 1)
        vh_ref[rows, DH:] = jnp.where(lane == 0, 1.0, 0.0).astype(BF16)
        return carry

    lax.fori_loop(0, S_FULL // KBLK, kstep, 0)

    def qstep(qb, carry):
        rows = pl.ds(qb * QBLK, QBLK)
        xb = x_ref[rows, :]
        qh = jnp.dot(xb, wq16, preferred_element_type=F32)
        qrot = (qh * cos_ref[rows, :]
                + jnp.dot(qh.astype(BF16), rmat16, preferred_element_type=F32)
                * sin_ref[rows, :])
        q16 = (qrot * (SCALE * LOG2E)).astype(BF16)
        s = lax.dot_general(
            q16, krot_ref[...], (((1,), (1,)), ((), ())),
            preferred_element_type=F32,
        )
        e = jnp.exp2(s).astype(BF16)
        ctxsum = jnp.dot(e, vh_ref[...], preferred_element_type=F32)
        ctx = ctxsum[:, :DH] * (1.0 / ctxsum[:, DH:DH + 1])
        contrib = jnp.dot(ctx.astype(BF16), wo16, preferred_element_type=F32)

        @pl.when(h == 0)
        def _():
            out_ref[rows, :] = contrib

        @pl.when(h != 0)
        def _():
            out_ref[rows, :] = out_ref[rows, :] + contrib

        return carry

    lax.fori_loop(0, S_FULL // QBLK, qstep, 0)


def _attention(x_full, Wq, Wk, Wv, Wo, cos, sin):
    return pl.pallas_call(
        _attn_body,
        grid=(HQ,),
        out_shape=jax.ShapeDtypeStruct((S_FULL, D), F32),
        in_specs=[
            pl.BlockSpec(memory_space=pltpu.VMEM),
            pl.BlockSpec((D, DH), lambda h: (0, h)),
            pl.BlockSpec((D, DH), lambda h: (0, h)),
            pl.BlockSpec((D, DH), lambda h: (0, h)),
            pl.BlockSpec((DH, D), lambda h: (h, 0)),
            pl.BlockSpec(memory_space=pltpu.VMEM),
            pl.BlockSpec(memory_space=pltpu.VMEM),
        ],
        out_specs=pl.BlockSpec(memory_space=pltpu.VMEM),
        scratch_shapes=[
            pltpu.VMEM((S_FULL, DH), jnp.bfloat16),
            pltpu.VMEM((S_FULL, 2 * DH), jnp.bfloat16),
        ],
        compiler_params=pltpu.CompilerParams(
            vmem_limit_bytes=64 * 1024 * 1024,
        ),
    )(x_full, Wq, Wk, Wv, Wo, cos, sin)


def _rs_body(p_ref, out_ref, recva_ref, recvb_ref,
             sa_sems, ra_sems, sb_sems, rb_sems):
    my = lax.axis_index("i")
    left = (my + N_DEV - 1) % N_DEV
    right = (my + 1) % N_DEV
    _neighbor_barrier(left, right)

    acc = recva_ref.dtype

    def _send(src, dst, ssem, rsem, dev):
        rdma = pltpu.make_async_remote_copy(
            src_ref=src, dst_ref=dst, send_sem=ssem, recv_sem=rsem,
            device_id=(dev,), device_id_type=pl.DeviceIdType.MESH,
        )
        rdma.start()
        return rdma

    for s in range(N_DEV - 1):
        ca = (my + N_DEV - 1 - s) % N_DEV
        cb = (my + 1 + s) % N_DEV
        sla = pl.ds(ca * S_SHARD, HALF)
        slb = pl.ds(cb * S_SHARD + HALF, HALF)
        if s == 0:
            src_a = p_ref.at[sla, :]
            src_b = p_ref.at[slb, :]
        else:
            recva_ref[s - 1, :, :] = (
                recva_ref[s - 1, :, :].astype(F32) + p_ref[sla, :].astype(F32)
            ).astype(acc)
            recvb_ref[s - 1, :, :] = (
                recvb_ref[s - 1, :, :].astype(F32) + p_ref[slb, :].astype(F32)
            ).astype(acc)
            src_a = recva_ref.at[s - 1]
            src_b = recvb_ref.at[s - 1]
        rda = _send(src_a, recva_ref.at[s], sa_sems.at[s], ra_sems.at[s], right)
        rdb = _send(src_b, recvb_ref.at[s], sb_sems.at[s], rb_sems.at[s], left)
        rda.wait()
        rdb.wait()

    last = N_DEV - 2
    out_ref[0, :HALF, :] = (
        recva_ref[last, :, :].astype(F32)
        + p_ref[pl.ds(my * S_SHARD, HALF), :].astype(F32)
    )
    out_ref[0, HALF:, :] = (
        recvb_ref[last, :, :].astype(F32)
        + p_ref[pl.ds(my * S_SHARD + HALF, HALF), :].astype(F32)
    )


def _reduce_scatter(partial):
    return pl.pallas_call(
        _rs_body,
        out_shape=jax.ShapeDtypeStruct((1, S_SHARD, D), F32),
        in_specs=[pl.BlockSpec(memory_space=pltpu.VMEM)],
        out_specs=pl.BlockSpec(memory_space=pltpu.VMEM),
        scratch_shapes=[
            pltpu.VMEM((N_DEV - 1, HALF, D), partial.dtype),
            pltpu.VMEM((N_DEV - 1, HALF, D), partial.dtype),
            pltpu.SemaphoreType.DMA((N_DEV - 1,)),
            pltpu.SemaphoreType.DMA((N_DEV - 1,)),
            pltpu.SemaphoreType.DMA((N_DEV - 1,)),
            pltpu.SemaphoreType.DMA((N_DEV - 1,)),
        ],
        compiler_params=pltpu.CompilerParams(
            collective_id=1, vmem_limit_bytes=64 * 1024 * 1024,
        ),
    )(partial)


def kernel(x, Wq, Wk, Wv, Wo):
    inv = 1.0 / (10000.0 ** (jnp.arange(0, DH, 2, dtype=F32) / DH))
    pos = jnp.arange(S_FULL, dtype=F32)[:, None] * inv[None, :]
    cos = jnp.repeat(jnp.cos(pos), 2, axis=-1)
    sin = jnp.repeat(jnp.sin(pos), 2, axis=-1)

    x_full = _ring_all_gather(x.astype(jnp.bfloat16))
    partial = _attention(x_full, Wq, Wk, Wv, Wo, cos, sin)
    return _reduce_scatter(partial.astype(jnp.bfloat16))


# device time: 482755 ns/iter; 1.4472x vs baseline; 1.0200x over previous
import jax
import jax.numpy as jnp
from jax import lax
from jax.experimental import pallas as pl
from jax.experimental.pallas import tpu as pltpu

N_DEV = 4
S_SHARD = 1024
S_FULL = 4096
D = 1024
HQ = 8
DH = 128
QBLK = 1024
KBLK = 1024
SCALE = 0.08838834764831843
LOG2E = 1.4426950408889634
F32 = jnp.float32


def _neighbor_barrier(left, right):
    barrier = pltpu.get_barrier_semaphore()
    for nbr in (left, right):
        pl.semaphore_signal(
            barrier, inc=1,
            device_id=(nbr,), device_id_type=pl.DeviceIdType.MESH,
        )
    pl.semaphore_wait(barrier, 2)


HALF = S_SHARD // 2


def _ag_body(x_ref, out_ref, sa_sems, ra_sems, sb_sems, rb_sems):
    my = lax.axis_index("i")
    left = (my + N_DEV - 1) % N_DEV
    right = (my + 1) % N_DEV
    _neighbor_barrier(left, right)

    out_ref[pl.ds(my * S_SHARD, S_SHARD), :] = x_ref[0]

    for h in range(N_DEV - 1):
        oa = (my + N_DEV - h) % N_DEV
        ob = (my + h) % N_DEV
        sla = pl.ds(oa * S_SHARD, HALF)
        slb = pl.ds(ob * S_SHARD + HALF, HALF)
        ra = pltpu.make_async_remote_copy(
            src_ref=out_ref.at[sla, :], dst_ref=out_ref.at[sla, :],
            send_sem=sa_sems.at[h], recv_sem=ra_sems.at[h],
            device_id=(right,), device_id_type=pl.DeviceIdType.MESH,
        )
        rb = pltpu.make_async_remote_copy(
            src_ref=out_ref.at[slb, :], dst_ref=out_ref.at[slb, :],
            send_sem=sb_sems.at[h], recv_sem=rb_sems.at[h],
            device_id=(left,), device_id_type=pl.DeviceIdType.MESH,
        )
        ra.start()
        rb.start()
        ra.wait()
        rb.wait()


def _ring_all_gather(x):
    return pl.pallas_call(
        _ag_body,
        out_shape=jax.ShapeDtypeStruct((S_FULL, D), x.dtype),
        in_specs=[pl.BlockSpec(memory_space=pltpu.VMEM)],
        out_specs=pl.BlockSpec(memory_space=pltpu.VMEM),
        scratch_shapes=[
            pltpu.SemaphoreType.DMA((N_DEV - 1,)),
            pltpu.SemaphoreType.DMA((N_DEV - 1,)),
            pltpu.SemaphoreType.DMA((N_DEV - 1,)),
            pltpu.SemaphoreType.DMA((N_DEV - 1,)),
        ],
        compiler_params=pltpu.CompilerParams(
            collective_id=0, vmem_limit_bytes=64 * 1024 * 1024,
        ),
    )(x)


def _attn_body(x_ref, wq_ref, wk_ref, wv_ref, wo_ref, cos_ref, sin_ref,
               out_ref, krot_ref, vh_ref):
    h = pl.program_id(0)

    ii = lax.broadcasted_iota(jnp.int32, (DH, DH), 0)
    jj = lax.broadcasted_iota(jnp.int32, (DH, DH), 1)
    rmat = jnp.where(
        (ii % 2 == 0) & (jj == ii + 1), 1.0,
        jnp.where((ii % 2 == 1) & (jj == ii - 1), -1.0, 0.0),
    ).astype(F32)

    BF16 = jnp.bfloat16
    wk16 = wk_ref[...].astype(BF16)
    wv16 = wv_ref[...].astype(BF16)
    wq16 = wq_ref[...].astype(BF16)
    wo16 = wo_ref[...].astype(BF16)
    rmat16 = rmat.astype(BF16)

    def kstep(c, carry):
        rows = pl.ds(c * KBLK, KBLK)
        xb = x_ref[rows, :]
        kh = jnp.dot(xb, wk16, preferred_element_type=F32)
        krot_ref[rows, :] = (
            kh * cos_ref[rows, :]
            + jnp.dot(kh.astype(BF16), rmat16, preferred_element_type=F32)
            * sin_ref[rows, :]
        ).astype(BF16)
        vh_ref[rows, :DH] = jnp.dot(
            xb, wv16, preferred_element_type=F32
        ).astype(BF16)
        lane = lax.broadcasted_iota(jnp.int32, (KBLK, DH), 1)
        vh_ref[rows, DH:] = jnp.where(lane == 0, 1.0, 0.0).astype(BF16)
        return carry

    lax.fori_loop(0, S_FULL // KBLK, kstep, 0)

    def qstep(qb, carry):
        rows = pl.ds(qb * QBLK, QBLK)
        xb = x_ref[rows, :]
        qh = jnp.dot(xb, wq16, preferred_element_type=F32)
        qrot = (qh * cos_ref[rows, :]
                + jnp.dot(qh.astype(BF16), rmat16, preferred_element_type=F32)
                * sin_ref[rows, :])
        q16 = (qrot * (SCALE * LOG2E)).astype(BF16)
        s = lax.dot_general(
            q16, krot_ref[...], (((1,), (1,)), ((), ())),
            preferred_element_type=F32,
        )
        e = jnp.exp2(s).astype(BF16)
        ctxsum = jnp.dot(e, vh_ref[...], preferred_element_type=F32)
        ctx = ctxsum[:, :DH] * (1.0 / ctxsum[:, DH:DH + 1])
        contrib = jnp.dot(ctx.astype(BF16), wo16, preferred_element_type=F32)

        @pl.when(h == 0)
        def _():
            out_ref[rows, :] = contrib

        @pl.when(h != 0)
        def _():
            out_ref[rows, :] = out_ref[rows, :] + contrib

        return carry

    lax.fori_loop(0, S_FULL // QBLK, qstep, 0)


def _attention(x_full, Wq, Wk, Wv, Wo, cos, sin):
    return pl.pallas_call(
        _attn_body,
        grid=(HQ,),
        out_shape=jax.ShapeDtypeStruct((S_FULL, D), F32),
        in_specs=[
            pl.BlockSpec(memory_space=pltpu.VMEM),
            pl.BlockSpec((D, DH), lambda h: (0, h)),
            pl.BlockSpec((D, DH), lambda h: (0, h)),
            pl.BlockSpec((D, DH), lambda h: (0, h)),
            pl.BlockSpec((DH, D), lambda h: (h, 0)),
            pl.BlockSpec(memory_space=pltpu.VMEM),
            pl.BlockSpec(memory_space=pltpu.VMEM),
        ],
        out_specs=pl.BlockSpec(memory_space=pltpu.VMEM),
        scratch_shapes=[
            pltpu.VMEM((S_FULL, DH), jnp.bfloat16),
            pltpu.VMEM((S_FULL, 2 * DH), jnp.bfloat16),
        ],
        compiler_params=pltpu.CompilerParams(
            vmem_limit_bytes=64 * 1024 * 1024,
        ),
    )(x_full, Wq, Wk, Wv, Wo, cos, sin)


def _rs_body(p_ref, out_ref, recva_ref, recvb_ref,
             sa_sems, ra_sems, sb_sems, rb_sems):
    my = lax.axis_index("i")
    left = (my + N_DEV - 1) % N_DEV
    right = (my + 1) % N_DEV
    _neighbor_barrier(left, right)

    acc = recva_ref.dtype

    def _send(src, dst, ssem, rsem, dev):
        rdma = pltpu.make_async_remote_copy(
            src_ref=src, dst_ref=dst, send_sem=ssem, recv_sem=rsem,
            device_id=(dev,), device_id_type=pl.DeviceIdType.MESH,
        )
        rdma.start()
        return rdma

    for s in range(N_DEV - 1):
        ca = (my + N_DEV - 1 - s) % N_DEV
        cb = (my + 1 + s) % N_DEV
        sla = pl.ds(ca * S_SHARD, HALF)
        slb = pl.ds(cb * S_SHARD + HALF, HALF)
        if s == 0:
            src_a = p_ref.at[sla, :]
            src_b = p_ref.at[slb, :]
        else:
            recva_ref[s - 1, :, :] = (
                recva_ref[s - 1, :, :].astype(F32) + p_ref[sla, :].astype(F32)
            ).astype(acc)
            recvb_ref[s - 1, :, :] = (
                recvb_ref[s - 1, :, :].astype(F32) + p_ref[slb, :].astype(F32)
            ).astype(acc)
            src_a = recva_ref.at[s - 1]
            src_b = recvb_ref.at[s - 1]
        rda = _send(src_a, recva_ref.at[s], sa_sems.at[s], ra_sems.at[s], right)
        rdb = _send(src_b, recvb_ref.at[s], sb_sems.at[s], rb_sems.at[s], left)
        rda.wait()
        rdb.wait()

    last = N_DEV - 2
    out_ref[0, :HALF, :] = (
        recva_ref[last, :, :].astype(F32)
        + p_ref[pl.ds(my * S_SHARD, HALF), :].astype(F32)
    )
    out_ref[0, HALF:, :] = (
        recvb_ref[last, :, :].astype(F32)
        + p_ref[pl.ds(my * S_SHARD + HALF, HALF), :].astype(F32)
    )


def _reduce_scatter(partial):
    return pl.pallas_call(
        _rs_body,
        out_shape=jax.ShapeDtypeStruct((1, S_SHARD, D), F32),
        in_specs=[pl.BlockSpec(memory_space=pltpu.VMEM)],
        out_specs=pl.BlockSpec(memory_space=pltpu.VMEM),
        scratch_shapes=[
            pltpu.VMEM((N_DEV - 1, HALF, D), partial.dtype),
            pltpu.VMEM((N_DEV - 1, HALF, D), partial.dtype),
            pltpu.SemaphoreType.DMA((N_DEV - 1,)),
            pltpu.SemaphoreType.DMA((N_DEV - 1,)),
            pltpu.SemaphoreType.DMA((N_DEV - 1,)),
            pltpu.SemaphoreType.DMA((N_DEV - 1,)),
        ],
        compiler_params=pltpu.CompilerParams(
            collective_id=1, vmem_limit_bytes=64 * 1024 * 1024,
        ),
    )(partial)


def kernel(x, Wq, Wk, Wv, Wo):
    inv = 1.0 / (10000.0 ** (jnp.arange(0, DH, 2, dtype=F32) / DH))
    pos = jnp.arange(S_FULL, dtype=F32)[:, None] * inv[None, :]
    cos = jnp.repeat(jnp.cos(pos), 2, axis=-1)
    sin = jnp.repeat(jnp.sin(pos), 2, axis=-1)

    x_full = _ring_all_gather(x.astype(jnp.bfloat16))
    partial = _attention(x_full, Wq, Wk, Wv, Wo, cos, sin)
    return _reduce_scatter(partial.astype(jnp.bfloat16))
